# baseline (speedup 1.0000x reference)
import functools

import jax
import jax.numpy as jnp
from jax import lax
from jax.experimental import pallas as pl
from jax.experimental.pallas import tpu as pltpu
from jax.experimental.pallas import tpu_sc as plsc

EPS = 1e-12
CHUNK = 32


def _emb_rmsnorm_sc(ids, pids, word_table, pos_table, norm_weight):
    N = ids.shape[0]
    D = word_table.shape[1]
    info = plsc.get_sparse_core_info()
    NC, NS, L = info.num_cores, info.num_subcores, info.num_lanes
    NW = NC * NS
    per_w = N // NW
    n_ch = per_w // CHUNK
    nvec = D // L

    mesh = plsc.VectorSubcoreMesh(core_axis_name="c", subcore_axis_name="s")

    @functools.partial(
        pl.kernel,
        mesh=mesh,
        out_type=jax.ShapeDtypeStruct((N, D), jnp.float32),
        compiler_params=pltpu.CompilerParams(needs_layout_passes=False),
        scratch_types=[
            pltpu.VMEM((CHUNK,), jnp.int32),
            pltpu.VMEM((CHUNK,), jnp.int32),
            pltpu.VMEM((CHUNK, D), jnp.float32),
            pltpu.VMEM((CHUNK, D), jnp.float32),
            pltpu.VMEM((D,), jnp.float32),
            pltpu.SemaphoreType.DMA,
            pltpu.SemaphoreType.DMA,
        ],
    )
    def emb_kernel(ids_hbm, pid_hbm, wt_hbm, pt_hbm, nw_hbm, out_hbm,
                   widx, pidxv, wbuf, pbuf, nwv, sem_w, sem_p):
        wid = lax.axis_index("s") * NC + lax.axis_index("c")
        base = wid * per_w
        pltpu.sync_copy(nw_hbm, nwv)
        for c in range(n_ch):
            off = base + c * CHUNK
            pltpu.sync_copy(ids_hbm.at[pl.ds(off, CHUNK)], widx)
            pltpu.sync_copy(pid_hbm.at[pl.ds(off, CHUNK)], pidxv)
            cw = pltpu.async_copy(wt_hbm.at[widx], wbuf, sem_w)
            cp = pltpu.async_copy(pt_hbm.at[pidxv], pbuf, sem_p)
            cw.wait()
            cp.wait()

            def body(t, carry):
                vacc = jnp.zeros((L,), jnp.float32)
                for j in range(nvec):
                    sl = pl.ds(j * L, L)
                    v = wbuf[t, sl] + pbuf[t, sl]
                    wbuf[t, sl] = v * nwv[sl]
                    vacc = vacc + v * v
                total = jnp.sum(vacc)
                dv = jnp.broadcast_to(total * (1.0 / D) + EPS, (L,))
                bits = plsc.bitcast(dv, jnp.int32)
                magic = jnp.full((L,), 0x5F3759DF, dtype=jnp.int32)
                one = jnp.full((L,), 1, dtype=jnp.int32)
                y = plsc.bitcast(magic - lax.shift_right_logical(bits, one),
                                 jnp.float32)
                for _ in range(3):
                    y = y * (1.5 - 0.5 * dv * y * y)
                for j in range(nvec):
                    sl = pl.ds(j * L, L)
                    wbuf[t, sl] = wbuf[t, sl] * y
                return carry

            lax.fori_loop(0, CHUNK, body, 0)
            pltpu.sync_copy(wbuf, out_hbm.at[pl.ds(off, CHUNK)])

    return emb_kernel(ids, pids, word_table, pos_table, norm_weight)


def kernel(input_ids, position_idcs, word_table, pos_table, norm_weight):
    B, S = input_ids.shape
    D = word_table.shape[1]
    ids = input_ids.reshape(B * S).astype(jnp.int32)
    pids = position_idcs.reshape(B * S).astype(jnp.int32)
    out = _emb_rmsnorm_sc(ids, pids, word_table.astype(jnp.float32),
                          pos_table.astype(jnp.float32),
                          norm_weight.astype(jnp.float32))
    return out.reshape(B, S, D)


# speedup vs baseline: 1.1250x; 1.1250x over previous
import functools

import jax
import jax.numpy as jnp
from jax import lax
from jax.experimental import pallas as pl
from jax.experimental.pallas import tpu as pltpu
from jax.experimental.pallas import tpu_sc as plsc

EPS = 1e-12
CHUNK = 32


def _emb_rmsnorm_sc(ids, pids, word_table, pos_table, norm_weight):
    NW_, n_ch, _ = ids.shape
    D = word_table.shape[1]
    info = plsc.get_sparse_core_info()
    NC, NS, L = info.num_cores, info.num_subcores, info.num_lanes
    NW = NC * NS
    assert NW_ == NW
    N = NW * n_ch * CHUNK
    per_w = n_ch * CHUNK
    nvec = D // L

    mesh = plsc.VectorSubcoreMesh(core_axis_name="c", subcore_axis_name="s")

    @functools.partial(
        pl.kernel,
        mesh=mesh,
        out_type=jax.ShapeDtypeStruct((N, D), jnp.float32),
        compiler_params=pltpu.CompilerParams(needs_layout_passes=False),
        scratch_types=[
            pltpu.VMEM((n_ch, CHUNK), jnp.int32),
            pltpu.VMEM((n_ch, CHUNK), jnp.int32),
            pltpu.VMEM((2, CHUNK, D), jnp.float32),
            pltpu.VMEM((2, CHUNK, D), jnp.float32),
            pltpu.VMEM((D,), jnp.float32),
            pltpu.SemaphoreType.DMA,
            pltpu.SemaphoreType.DMA,
            pltpu.SemaphoreType.DMA,
            pltpu.SemaphoreType.DMA,
            pltpu.SemaphoreType.DMA,
            pltpu.SemaphoreType.DMA,
        ],
    )
    def emb_kernel(ids_hbm, pid_hbm, wt_hbm, pt_hbm, nw_hbm, out_hbm,
                   widx, pidxv, wbuf, pbuf, nwv,
                   sw0, sw1, sp0, sp1, so0, so1):
        wid = lax.axis_index("s") * NC + lax.axis_index("c")
        base = wid * per_w
        sems_w = (sw0, sw1)
        sems_p = (sp0, sp1)
        sems_o = (so0, so1)

        pltpu.sync_copy(nw_hbm, nwv)
        pltpu.sync_copy(ids_hbm.at[wid], widx)
        pltpu.sync_copy(pid_hbm.at[wid], pidxv)

        def gather(c):
            b = c & 1
            cw = pltpu.async_copy(wt_hbm.at[widx.at[c]], wbuf.at[b], sems_w[b])
            cp = pltpu.async_copy(pt_hbm.at[pidxv.at[c]], pbuf.at[b], sems_p[b])
            return cw, cp

        def compute(b):
            def body(t, carry):
                vacc = jnp.zeros((L,), jnp.float32)
                for j in range(nvec):
                    sl = pl.ds(j * L, L)
                    v = wbuf[b, t, sl] + pbuf[b, t, sl]
                    wbuf[b, t, sl] = v * nwv[sl]
                    vacc = vacc + v * v
                total = jnp.sum(vacc)
                dv = jnp.broadcast_to(total * (1.0 / D) + EPS, (L,))
                bits = plsc.bitcast(dv, jnp.int32)
                magic = jnp.full((L,), 0x5F3759DF, dtype=jnp.int32)
                one = jnp.full((L,), 1, dtype=jnp.int32)
                y = plsc.bitcast(magic - lax.shift_right_logical(bits, one),
                                 jnp.float32)
                for _ in range(2):
                    y = y * (1.5 - 0.5 * dv * y * y)
                for j in range(nvec):
                    sl = pl.ds(j * L, L)
                    wbuf[b, t, sl] = wbuf[b, t, sl] * y
                return carry

            lax.fori_loop(0, CHUNK, body, 0)

        pend = {0: gather(0)}
        out_pend = {}
        for c in range(n_ch):
            b = c & 1
            if c + 1 < n_ch:
                if c - 1 >= 0:
                    out_pend.pop(c - 1).wait()
                pend[c + 1] = gather(c + 1)
            cw, cp = pend.pop(c)
            cw.wait()
            cp.wait()
            compute(b)
            out_pend[c] = pltpu.async_copy(
                wbuf.at[b], out_hbm.at[pl.ds(base + c * CHUNK, CHUNK)],
                sems_o[b])
        out_pend.pop(n_ch - 2).wait()
        out_pend.pop(n_ch - 1).wait()

    return emb_kernel(ids, pids, word_table, pos_table, norm_weight)


def kernel(input_ids, position_idcs, word_table, pos_table, norm_weight):
    B, S = input_ids.shape
    D = word_table.shape[1]
    N = B * S
    NW = 32
    per_w = N // NW
    n_ch = per_w // CHUNK
    ids = input_ids.reshape(NW, n_ch, CHUNK).astype(jnp.int32)
    pids = position_idcs.reshape(NW, n_ch, CHUNK).astype(jnp.int32)
    out = _emb_rmsnorm_sc(ids, pids, word_table.astype(jnp.float32),
                          pos_table.astype(jnp.float32),
                          norm_weight.astype(jnp.float32))
    return out.reshape(B, S, D)


# speedup vs baseline: 1.4345x; 1.2752x over previous
import functools

import jax
import jax.numpy as jnp
from jax import lax
from jax.experimental import pallas as pl
from jax.experimental.pallas import tpu as pltpu
from jax.experimental.pallas import tpu_sc as plsc

EPS = 1e-12
CHUNK = 32


def _emb_rmsnorm_sc(ids, pids, word_table, pos_table, norm_weight):
    NW_, n_ch, _ = ids.shape
    D = word_table.shape[1]
    info = plsc.get_sparse_core_info()
    NC, NS, L = info.num_cores, info.num_subcores, info.num_lanes
    NW = NC * NS
    assert NW_ == NW
    N = NW * n_ch * CHUNK
    per_w = n_ch * CHUNK
    nvec = D // L

    mesh = plsc.VectorSubcoreMesh(core_axis_name="c", subcore_axis_name="s")

    @functools.partial(
        pl.kernel,
        mesh=mesh,
        out_type=jax.ShapeDtypeStruct((N, D), jnp.float32),
        compiler_params=pltpu.CompilerParams(needs_layout_passes=False),
        scratch_types=[
            pltpu.VMEM((n_ch, CHUNK), jnp.int32),
            pltpu.VMEM((n_ch, CHUNK), jnp.int32),
            pltpu.VMEM((2, CHUNK, D), jnp.float32),
            pltpu.VMEM((2, CHUNK, D), jnp.float32),
            pltpu.VMEM((D,), jnp.float32),
            pltpu.SemaphoreType.DMA,
            pltpu.SemaphoreType.DMA,
            pltpu.SemaphoreType.DMA,
            pltpu.SemaphoreType.DMA,
            pltpu.SemaphoreType.DMA,
            pltpu.SemaphoreType.DMA,
        ],
    )
    def emb_kernel(ids_hbm, pid_hbm, wt_hbm, pt_hbm, nw_hbm, out_hbm,
                   widx, pidxv, wbuf, pbuf, nwv,
                   sw0, sw1, sp0, sp1, so0, so1):
        wid = lax.axis_index("s") * NC + lax.axis_index("c")
        base = wid * per_w
        sems_w = (sw0, sw1)
        sems_p = (sp0, sp1)
        sems_o = (so0, so1)

        pltpu.sync_copy(nw_hbm, nwv)
        pltpu.sync_copy(ids_hbm.at[wid], widx)
        pltpu.sync_copy(pid_hbm.at[wid], pidxv)

        def gather(c):
            b = c & 1
            cw = pltpu.async_copy(wt_hbm.at[widx.at[c]], wbuf.at[b], sems_w[b])
            cp = pltpu.async_copy(pt_hbm.at[pidxv.at[c]], pbuf.at[b], sems_p[b])
            return cw, cp

        def compute(b):
            @plsc.parallel_loop(0, CHUNK, unroll=2)
            def body(t):
                accs = [jnp.zeros((L,), jnp.float32) for _ in range(4)]
                for j in range(nvec):
                    sl = pl.ds(j * L, L)
                    v = wbuf[b, t, sl] + pbuf[b, t, sl]
                    wbuf[b, t, sl] = v * nwv[sl]
                    accs[j & 3] = accs[j & 3] + v * v
                total = jnp.sum((accs[0] + accs[1]) + (accs[2] + accs[3]))
                dv = jnp.broadcast_to(total * (1.0 / D) + EPS, (L,))
                bits = plsc.bitcast(dv, jnp.int32)
                magic = jnp.full((L,), 0x5F3759DF, dtype=jnp.int32)
                one = jnp.full((L,), 1, dtype=jnp.int32)
                y = plsc.bitcast(magic - lax.shift_right_logical(bits, one),
                                 jnp.float32)
                for _ in range(2):
                    y = y * (1.5 - 0.5 * dv * y * y)
                for j in range(nvec):
                    sl = pl.ds(j * L, L)
                    wbuf[b, t, sl] = wbuf[b, t, sl] * y

        pend = {0: gather(0)}
        out_pend = {}
        for c in range(n_ch):
            b = c & 1
            if c + 1 < n_ch:
                if c - 1 >= 0:
                    out_pend.pop(c - 1).wait()
                pend[c + 1] = gather(c + 1)
            cw, cp = pend.pop(c)
            cw.wait()
            cp.wait()
            compute(b)
            out_pend[c] = pltpu.async_copy(
                wbuf.at[b], out_hbm.at[pl.ds(base + c * CHUNK, CHUNK)],
                sems_o[b])
        out_pend.pop(n_ch - 2).wait()
        out_pend.pop(n_ch - 1).wait()

    return emb_kernel(ids, pids, word_table, pos_table, norm_weight)


def kernel(input_ids, position_idcs, word_table, pos_table, norm_weight):
    B, S = input_ids.shape
    D = word_table.shape[1]
    N = B * S
    NW = 32
    per_w = N // NW
    n_ch = per_w // CHUNK
    ids = input_ids.reshape(NW, n_ch, CHUNK).astype(jnp.int32)
    pids = position_idcs.reshape(NW, n_ch, CHUNK).astype(jnp.int32)
    out = _emb_rmsnorm_sc(ids, pids, word_table.astype(jnp.float32),
                          pos_table.astype(jnp.float32),
                          norm_weight.astype(jnp.float32))
    return out.reshape(B, S, D)


# speedup vs baseline: 1.5725x; 1.0962x over previous
import functools

import jax
import jax.numpy as jnp
from jax import lax
from jax.experimental import pallas as pl
from jax.experimental.pallas import tpu as pltpu
from jax.experimental.pallas import tpu_sc as plsc

EPS = 1e-12
CHUNK = 32
NBUF = 2


def _emb_rmsnorm_sc(ids, pids, word_table, pos_table, norm_weight):
    NW_, n_ch, _ = ids.shape
    D = word_table.shape[1]
    info = plsc.get_sparse_core_info()
    NC, NS, L = info.num_cores, info.num_subcores, info.num_lanes
    NW = NC * NS
    assert NW_ == NW
    N = NW * n_ch * CHUNK
    per_w = n_ch * CHUNK
    nvec = D // L

    mesh = plsc.VectorSubcoreMesh(core_axis_name="c", subcore_axis_name="s")

    @functools.partial(
        pl.kernel,
        mesh=mesh,
        out_type=jax.ShapeDtypeStruct((N, D), jnp.float32),
        compiler_params=pltpu.CompilerParams(needs_layout_passes=False),
        scratch_types=[
            pltpu.VMEM((n_ch, CHUNK), jnp.int32),
            pltpu.VMEM((n_ch, CHUNK), jnp.int32),
            pltpu.VMEM((NBUF, CHUNK, D), jnp.float32),
            pltpu.VMEM((NBUF, CHUNK, D), jnp.float32),
            pltpu.VMEM((D,), jnp.float32),
            pltpu.SemaphoreType.DMA((NBUF,)),
            pltpu.SemaphoreType.DMA((NBUF,)),
            pltpu.SemaphoreType.DMA((NBUF,)),
        ],
    )
    def emb_kernel(ids_hbm, pid_hbm, wt_hbm, pt_hbm, nw_hbm, out_hbm,
                   widx, pidxv, wbuf, pbuf, nwv, semw, semp, semo):
        wid = lax.axis_index("s") * NC + lax.axis_index("c")
        base = wid * per_w

        pltpu.sync_copy(nw_hbm, nwv)
        pltpu.sync_copy(ids_hbm.at[wid], widx)
        pltpu.sync_copy(pid_hbm.at[wid], pidxv)

        def w_desc(c):
            b = lax.rem(c, NBUF)
            return pltpu.make_async_copy(wt_hbm.at[widx.at[c]], wbuf.at[b],
                                         semw.at[b])

        def p_desc(c):
            b = lax.rem(c, NBUF)
            return pltpu.make_async_copy(pt_hbm.at[pidxv.at[c]], pbuf.at[b],
                                         semp.at[b])

        def out_desc(c):
            b = lax.rem(c, NBUF)
            return pltpu.make_async_copy(
                wbuf.at[b], out_hbm.at[pl.ds(base + c * CHUNK, CHUNK)],
                semo.at[b])

        def gather(c):
            w_desc(c).start()
            p_desc(c).start()

        def compute(c):
            b = lax.rem(c, NBUF)

            @plsc.parallel_loop(0, CHUNK, unroll=2)
            def body(t):
                accs = [jnp.zeros((L,), jnp.float32) for _ in range(4)]
                for j in range(nvec):
                    sl = pl.ds(j * L, L)
                    v = wbuf[b, t, sl] + pbuf[b, t, sl]
                    wbuf[b, t, sl] = v * nwv[sl]
                    accs[j & 3] = accs[j & 3] + v * v
                total = jnp.sum((accs[0] + accs[1]) + (accs[2] + accs[3]))
                dv = jnp.broadcast_to(total * (1.0 / D) + EPS, (L,))
                bits = plsc.bitcast(dv, jnp.int32)
                magic = jnp.full((L,), 0x5F3759DF, dtype=jnp.int32)
                one = jnp.full((L,), 1, dtype=jnp.int32)
                y = plsc.bitcast(magic - lax.shift_right_logical(bits, one),
                                 jnp.float32)
                for _ in range(2):
                    y = y * (1.5 - 0.5 * dv * y * y)
                for j in range(nvec):
                    sl = pl.ds(j * L, L)
                    wbuf[b, t, sl] = wbuf[b, t, sl] * y

        gather(jnp.int32(0))

        def body(c, carry):
            @pl.when(c + 1 < n_ch)
            def _():
                @pl.when(c >= 1)
                def _():
                    out_desc(c - 1).wait()
                gather(c + 1)

            w_desc(c).wait()
            p_desc(c).wait()
            compute(c)
            out_desc(c).start()
            return carry

        lax.fori_loop(0, n_ch, body, 0)
        for c in range(max(n_ch - NBUF, 0), n_ch):
            out_desc(jnp.int32(c)).wait()

    return emb_kernel(ids, pids, word_table, pos_table, norm_weight)


def kernel(input_ids, position_idcs, word_table, pos_table, norm_weight):
    B, S = input_ids.shape
    D = word_table.shape[1]
    N = B * S
    NW = 32
    per_w = N // NW
    n_ch = per_w // CHUNK
    ids = input_ids.reshape(NW, n_ch, CHUNK).astype(jnp.int32)
    pids = position_idcs.reshape(NW, n_ch, CHUNK).astype(jnp.int32)
    out = _emb_rmsnorm_sc(ids, pids, word_table.astype(jnp.float32),
                          pos_table.astype(jnp.float32),
                          norm_weight.astype(jnp.float32))
    return out.reshape(B, S, D)
